# baseline (speedup 1.0000x reference)
import functools

import jax
import jax.numpy as jnp
from jax import lax
from jax.experimental import pallas as pl
from jax.experimental.pallas import tpu as pltpu
from jax.experimental.pallas import tpu_sc as plsc

NUM_TOTEMS = 100000
POS_DIM = 3
BATCH = 16384

NW = 32
NB = BATCH // NW
NE = NB * POS_DIM
CHUNK = 128
NCHUNK = NB // CHUNK


@functools.partial(
    pl.kernel,
    mesh=plsc.VectorSubcoreMesh(core_axis_name="c", subcore_axis_name="s"),
    out_type=jax.ShapeDtypeStruct((NW, NE), jnp.float32),
    compiler_params=pltpu.CompilerParams(
        use_tc_tiling_on_sc=False, needs_layout_passes=False),
    scratch_types=[
        pltpu.VMEM((NCHUNK, CHUNK), jnp.int32),
        pltpu.VMEM((NB, POS_DIM), jnp.float32),
        pltpu.VMEM((NB, POS_DIM), jnp.float32),
        pltpu.VMEM((NE,), jnp.float32),
        pltpu.SemaphoreType.DMA,
    ],
)
def _lookup_add(ids_hbm, ta_hbm, tb_hbm, out_hbm, idx_v, a_v, b_v, o_v, sem):
    wid = lax.axis_index("s") * 2 + lax.axis_index("c")
    pltpu.sync_copy(ids_hbm.at[pl.ds(wid * NCHUNK, NCHUNK)], idx_v)

    copies = []
    for c in range(NCHUNK):
        idx_c = idx_v.at[c]
        copies.append(pltpu.async_copy(
            ta_hbm.at[idx_c], a_v.at[pl.ds(c * CHUNK, CHUNK)], sem))
        copies.append(pltpu.async_copy(
            tb_hbm.at[idx_c], b_v.at[pl.ds(c * CHUNK, CHUNK)], sem))
    for cp in copies:
        cp.wait()

    lane = lax.iota(jnp.int32, 16)

    @pl.loop(0, NB // 16)
    def body(i):
        r = i * 16 + lane
        base = i * (16 * POS_DIM) + lane * POS_DIM
        for d in range(POS_DIM):
            dv = lane * 0 + d
            va = plsc.load_gather(a_v, [r, dv])
            vb = plsc.load_gather(b_v, [r, dv])
            plsc.store_scatter(o_v, [base + d], va + vb)

    pltpu.sync_copy(o_v, out_hbm.at[wid])


def kernel(totem_id, initial_totem_pos, totem_pos_residual):
    ids = totem_id.astype(jnp.int32).reshape(NW * NCHUNK, CHUNK)
    out = _lookup_add(ids, initial_totem_pos, totem_pos_residual)
    return out.reshape(BATCH, POS_DIM)


# speedup vs baseline: 6.3013x; 6.3013x over previous
import functools

import jax
import jax.numpy as jnp
from jax import lax
from jax.experimental import pallas as pl
from jax.experimental.pallas import tpu as pltpu
from jax.experimental.pallas import tpu_sc as plsc

NUM_TOTEMS = 100000
POS_DIM = 3
BATCH = 16384

NW = 32
NB = BATCH // NW
CHUNK = 128
NCHUNK = NB // CHUNK


@functools.partial(
    pl.kernel,
    mesh=plsc.VectorSubcoreMesh(core_axis_name="c", subcore_axis_name="s"),
    out_type=jax.ShapeDtypeStruct((POS_DIM, NW, NB), jnp.float32),
    compiler_params=pltpu.CompilerParams(
        use_tc_tiling_on_sc=False, needs_layout_passes=False),
    scratch_types=[
        pltpu.VMEM((NCHUNK, CHUNK), jnp.int32),
        pltpu.VMEM((POS_DIM, NB), jnp.float32),
        pltpu.VMEM((POS_DIM, NB), jnp.float32),
        pltpu.VMEM((POS_DIM, NB), jnp.float32),
        pltpu.SemaphoreType.DMA,
    ],
)
def _lookup_add(ids_hbm, ta0, ta1, ta2, tb0, tb1, tb2, out_hbm,
                idx_v, a_v, b_v, o_v, sem):
    wid = lax.axis_index("s") * 2 + lax.axis_index("c")
    pltpu.sync_copy(ids_hbm.at[pl.ds(wid * NCHUNK, NCHUNK)], idx_v)

    ta = (ta0, ta1, ta2)
    tb = (tb0, tb1, tb2)
    copies = []
    for d in range(POS_DIM):
        for c in range(NCHUNK):
            idx_c = idx_v.at[c]
            sl = pl.ds(c * CHUNK, CHUNK)
            copies.append(pltpu.async_copy(ta[d].at[idx_c], a_v.at[d, sl], sem))
            copies.append(pltpu.async_copy(tb[d].at[idx_c], b_v.at[d, sl], sem))
    for cp in copies:
        cp.wait()

    @pl.loop(0, NB // 16)
    def body(i):
        sl = pl.ds(i * 16, 16)
        for d in range(POS_DIM):
            o_v[d, sl] = a_v[d, sl] + b_v[d, sl]

    for d in range(POS_DIM):
        pltpu.sync_copy(o_v.at[d], out_hbm.at[d, wid])


def kernel(totem_id, initial_totem_pos, totem_pos_residual):
    ids = totem_id.astype(jnp.int32).reshape(NW * NCHUNK, CHUNK)
    planes_a = [initial_totem_pos[:, d] for d in range(POS_DIM)]
    planes_b = [totem_pos_residual[:, d] for d in range(POS_DIM)]
    out = _lookup_add(ids, *planes_a, *planes_b)
    return out.reshape(POS_DIM, BATCH).T


# speedup vs baseline: 7.8956x; 1.2530x over previous
import functools

import jax
import jax.numpy as jnp
from jax import lax
from jax.experimental import pallas as pl
from jax.experimental.pallas import tpu as pltpu
from jax.experimental.pallas import tpu_sc as plsc

NUM_TOTEMS = 100000
POS_DIM = 3
BATCH = 16384

NW = 32
NB = BATCH // NW
CHUNK = 128
NCHUNK = NB // CHUNK


@functools.partial(
    pl.kernel,
    mesh=plsc.VectorSubcoreMesh(core_axis_name="c", subcore_axis_name="s"),
    out_type=jax.ShapeDtypeStruct((POS_DIM, NW, NB), jnp.float32),
    compiler_params=pltpu.CompilerParams(
        use_tc_tiling_on_sc=False, needs_layout_passes=False),
    scratch_types=[
        pltpu.VMEM((NCHUNK, CHUNK), jnp.int32),
        pltpu.VMEM((POS_DIM, NB), jnp.float32),
        pltpu.SemaphoreType.DMA,
    ],
)
def _lookup(ids_hbm, ts0, ts1, ts2, out_hbm, idx_v, o_v, sem):
    wid = lax.axis_index("s") * 2 + lax.axis_index("c")
    pltpu.sync_copy(ids_hbm.at[pl.ds(wid * NCHUNK, NCHUNK)], idx_v)

    ts = (ts0, ts1, ts2)
    copies = []
    for d in range(POS_DIM):
        for c in range(NCHUNK):
            copies.append(pltpu.async_copy(
                ts[d].at[idx_v.at[c]],
                o_v.at[d, pl.ds(c * CHUNK, CHUNK)], sem))
    for cp in copies:
        cp.wait()

    for d in range(POS_DIM):
        pltpu.sync_copy(o_v.at[d], out_hbm.at[d, wid])


def kernel(totem_id, initial_totem_pos, totem_pos_residual):
    ids = totem_id.astype(jnp.int32).reshape(NW * NCHUNK, CHUNK)
    summed = initial_totem_pos + totem_pos_residual
    planes = [summed[:, d] for d in range(POS_DIM)]
    out = _lookup(ids, *planes)
    return out.reshape(POS_DIM, BATCH).T
